# baseline (speedup 1.0000x reference)
import jax
import jax.numpy as jnp
from jax.experimental import pallas as pl
from jax.experimental.pallas import tpu as pltpu

B = 8
D_MODEL = 1024
VOCAB = 100000
BV = 4000
NB = VOCAB // BV
NEG = -1e30


def _body(msg_ref, wt_ref, b_ref, ns_ref, lp_ref, ent_ref,
          m_ref, s_ref, t_ref, idx_ref):
    i = pl.program_id(0)

    @pl.when(i == 0)
    def _init():
        m_ref[...] = jnp.full((B, 1), NEG, jnp.float32)
        s_ref[...] = jnp.zeros((B, 1), jnp.float32)
        t_ref[...] = jnp.zeros((B, 1), jnp.float32)
        idx_ref[...] = jnp.zeros((B, 1), jnp.int32)

    logits = jax.lax.dot_general(
        msg_ref[...], wt_ref[...], (((1,), (1,)), ((), ())),
        preferred_element_type=jnp.float32)
    logits = logits + b_ref[0]

    col = i * BV + jax.lax.broadcasted_iota(jnp.int32, (B, BV), 1)

    bmax = jnp.max(logits, axis=1, keepdims=True)
    cand = jnp.where(logits == bmax, col, jnp.int32(2**31 - 1))
    bidx = jnp.min(cand, axis=1, keepdims=True)

    m_old = m_ref[...]
    new_m = jnp.maximum(m_old, bmax)
    e = jnp.exp(logits - new_m)
    scale = jnp.exp(m_old - new_m)
    s_ref[...] = s_ref[...] * scale + jnp.sum(e, axis=1, keepdims=True)
    t_ref[...] = t_ref[...] * scale + jnp.sum(logits * e, axis=1, keepdims=True)
    m_ref[...] = new_m
    idx_ref[...] = jnp.where(bmax > m_old, bidx, idx_ref[...])

    @pl.when(i == NB - 1)
    def _fin():
        m = m_ref[...]
        s = s_ref[...]
        lse = m + jnp.log(s)
        ns_ref[...] = idx_ref[...]
        lp_ref[...] = m - lse
        ent_ref[...] = lse - t_ref[...] / s


@jax.jit
def kernel(message, W, b):
    wt = W.T
    b2 = b.reshape(NB, 1, BV)
    ns, lp, ent = pl.pallas_call(
        _body,
        grid=(NB,),
        in_specs=[
            pl.BlockSpec((B, D_MODEL), lambda i: (0, 0)),
            pl.BlockSpec((BV, D_MODEL), lambda i: (i, 0)),
            pl.BlockSpec((1, 1, BV), lambda i: (i, 0, 0)),
        ],
        out_specs=[
            pl.BlockSpec((B, 1), lambda i: (0, 0)),
            pl.BlockSpec((B, 1), lambda i: (0, 0)),
            pl.BlockSpec((B, 1), lambda i: (0, 0)),
        ],
        out_shape=[
            jax.ShapeDtypeStruct((B, 1), jnp.int32),
            jax.ShapeDtypeStruct((B, 1), jnp.float32),
            jax.ShapeDtypeStruct((B, 1), jnp.float32),
        ],
        scratch_shapes=[
            pltpu.VMEM((B, 1), jnp.float32),
            pltpu.VMEM((B, 1), jnp.float32),
            pltpu.VMEM((B, 1), jnp.float32),
            pltpu.VMEM((B, 1), jnp.int32),
        ],
    )(message, wt, b2)
    return ns[:, 0], lp[:, 0], ent[:, 0]


# speedup vs baseline: 1.0352x; 1.0352x over previous
import jax
import jax.numpy as jnp
from jax.experimental import pallas as pl
from jax.experimental.pallas import tpu as pltpu

B = 8
D_MODEL = 1024
VOCAB = 100000
BV = 4096
NEG = -1e30


def _body(msg_ref, wt_ref, b_ref, ns_ref, lp_ref, ent_ref,
          m_ref, s_ref, t_ref, idx_ref):
    i = pl.program_id(0)
    nb = pl.num_programs(0)

    @pl.when(i == 0)
    def _init():
        m_ref[...] = jnp.full((B, 1), NEG, jnp.float32)
        s_ref[...] = jnp.zeros((B, 1), jnp.float32)
        t_ref[...] = jnp.zeros((B, 1), jnp.float32)
        idx_ref[...] = jnp.zeros((B, 1), jnp.int32)

    logits = jax.lax.dot_general(
        msg_ref[...], wt_ref[...], (((1,), (1,)), ((), ())),
        preferred_element_type=jnp.float32)
    logits = logits + b_ref[...]

    col = i * BV + jax.lax.broadcasted_iota(jnp.int32, (B, BV), 1)
    valid = col < VOCAB
    logits = jnp.where(valid, logits, NEG)

    bmax = jnp.max(logits, axis=1, keepdims=True)
    cand = jnp.where(logits == bmax, col, jnp.int32(2**31 - 1))
    bidx = jnp.min(cand, axis=1, keepdims=True)

    m_old = m_ref[...]
    new_m = jnp.maximum(m_old, bmax)
    e = jnp.exp(logits - new_m)
    scale = jnp.exp(m_old - new_m)
    s_ref[...] = s_ref[...] * scale + jnp.sum(e, axis=1, keepdims=True)
    t_ref[...] = t_ref[...] * scale + jnp.sum(logits * e, axis=1, keepdims=True)
    m_ref[...] = new_m
    idx_ref[...] = jnp.where(bmax > m_old, bidx, idx_ref[...])

    @pl.when(i == nb - 1)
    def _fin():
        m = m_ref[...]
        s = s_ref[...]
        lse = m + jnp.log(s)
        ns_ref[...] = idx_ref[...]
        lp_ref[...] = m - lse
        ent_ref[...] = lse - t_ref[...] / s


@jax.jit
def kernel(message, W, b):
    nb = pl.cdiv(VOCAB, BV)
    wt = W.T
    ns, lp, ent = pl.pallas_call(
        _body,
        grid=(nb,),
        in_specs=[
            pl.BlockSpec((B, D_MODEL), lambda i: (0, 0)),
            pl.BlockSpec((BV, D_MODEL), lambda i: (i, 0)),
            pl.BlockSpec((BV,), lambda i: (i,)),
        ],
        out_specs=[
            pl.BlockSpec((B, 1), lambda i: (0, 0)),
            pl.BlockSpec((B, 1), lambda i: (0, 0)),
            pl.BlockSpec((B, 1), lambda i: (0, 0)),
        ],
        out_shape=[
            jax.ShapeDtypeStruct((B, 1), jnp.int32),
            jax.ShapeDtypeStruct((B, 1), jnp.float32),
            jax.ShapeDtypeStruct((B, 1), jnp.float32),
        ],
        scratch_shapes=[
            pltpu.VMEM((B, 1), jnp.float32),
            pltpu.VMEM((B, 1), jnp.float32),
            pltpu.VMEM((B, 1), jnp.float32),
            pltpu.VMEM((B, 1), jnp.int32),
        ],
    )(message, wt, b)
    return ns[:, 0], lp[:, 0], ent[:, 0]


# speedup vs baseline: 1.0392x; 1.0039x over previous
import jax
import jax.numpy as jnp
from jax.experimental import pallas as pl
from jax.experimental.pallas import tpu as pltpu

B = 8
D_MODEL = 1024
VOCAB = 100000
BV = 4096
_NB = (VOCAB + BV - 1) // BV
NEG = -1e30


def _blk(i):
    return jnp.where(i == 0, _NB - 1, i - 1)


def _body(msg_ref, wt_ref, b_ref, ns_ref, lp_ref, ent_ref,
          m_ref, s_ref, t_ref, idx_ref):
    i = pl.program_id(0)
    nb = pl.num_programs(0)

    @pl.when(i == 0)
    def _init():
        m_ref[...] = jnp.full((B, 1), NEG, jnp.float32)
        s_ref[...] = jnp.zeros((B, 1), jnp.float32)
        t_ref[...] = jnp.zeros((B, 1), jnp.float32)
        idx_ref[...] = jnp.zeros((B, 1), jnp.int32)

    logits = jax.lax.dot_general(
        msg_ref[...], wt_ref[...], (((1,), (1,)), ((), ())),
        preferred_element_type=jnp.float32)
    logits = logits + b_ref[...]

    col = _blk(i) * BV + jax.lax.broadcasted_iota(jnp.int32, (B, BV), 1)
    valid = col < VOCAB
    logits = jnp.where(valid, logits, NEG)

    bmax = jnp.max(logits, axis=1, keepdims=True)
    cand = jnp.where(logits == bmax, col, jnp.int32(2**31 - 1))
    bidx = jnp.min(cand, axis=1, keepdims=True)

    m_old = m_ref[...]
    new_m = jnp.maximum(m_old, bmax)
    e = jnp.exp(logits - new_m)
    scale = jnp.exp(m_old - new_m)
    s_ref[...] = s_ref[...] * scale + jnp.sum(e, axis=1, keepdims=True)
    t_ref[...] = t_ref[...] * scale + jnp.sum(logits * e, axis=1, keepdims=True)
    m_ref[...] = new_m
    better = (bmax > m_old) | ((bmax == m_old) & (bidx < idx_ref[...]))
    idx_ref[...] = jnp.where(better, bidx, idx_ref[...])

    @pl.when(i == nb - 1)
    def _fin():
        m = m_ref[...]
        s = s_ref[...]
        lse = m + jnp.log(s)
        ns_ref[...] = idx_ref[...]
        lp_ref[...] = m - lse
        ent_ref[...] = lse - t_ref[...] / s


@jax.jit
def kernel(message, W, b):
    nb = pl.cdiv(VOCAB, BV)
    wt = W.T
    ns, lp, ent = pl.pallas_call(
        _body,
        grid=(nb,),
        in_specs=[
            pl.BlockSpec((B, D_MODEL), lambda i: (0, 0)),
            pl.BlockSpec((BV, D_MODEL), lambda i: (_blk(i), 0)),
            pl.BlockSpec((BV,), lambda i: (_blk(i),)),
        ],
        out_specs=[
            pl.BlockSpec((B, 1), lambda i: (0, 0)),
            pl.BlockSpec((B, 1), lambda i: (0, 0)),
            pl.BlockSpec((B, 1), lambda i: (0, 0)),
        ],
        out_shape=[
            jax.ShapeDtypeStruct((B, 1), jnp.int32),
            jax.ShapeDtypeStruct((B, 1), jnp.float32),
            jax.ShapeDtypeStruct((B, 1), jnp.float32),
        ],
        scratch_shapes=[
            pltpu.VMEM((B, 1), jnp.float32),
            pltpu.VMEM((B, 1), jnp.float32),
            pltpu.VMEM((B, 1), jnp.float32),
            pltpu.VMEM((B, 1), jnp.int32),
        ],
    )(message, wt, b)
    return ns[:, 0], lp[:, 0], ent[:, 0]


# speedup vs baseline: 1.0484x; 1.0088x over previous
import jax
import jax.numpy as jnp
from jax.experimental import pallas as pl
from jax.experimental.pallas import tpu as pltpu

B = 8
D_MODEL = 1024
VOCAB = 100000
BV = 3072
_NB = (VOCAB + BV - 1) // BV
NEG = -1e30


def _blk(i):
    return jnp.where(i == 0, _NB - 1, i - 1)


def _body(msg_ref, wt_ref, b_ref, ns_ref, lp_ref, ent_ref,
          m_ref, s_ref, t_ref, idx_ref):
    i = pl.program_id(0)
    nb = pl.num_programs(0)

    @pl.when(i == 0)
    def _init():
        m_ref[...] = jnp.full((B, 1), NEG, jnp.float32)
        s_ref[...] = jnp.zeros((B, 1), jnp.float32)
        t_ref[...] = jnp.zeros((B, 1), jnp.float32)
        idx_ref[...] = jnp.zeros((B, 1), jnp.int32)

    logits = jax.lax.dot_general(
        msg_ref[...], wt_ref[...], (((1,), (1,)), ((), ())),
        preferred_element_type=jnp.float32)
    logits = logits + b_ref[...]

    col = _blk(i) * BV + jax.lax.broadcasted_iota(jnp.int32, (B, BV), 1)
    valid = col < VOCAB
    logits = jnp.where(valid, logits, NEG)

    bmax = jnp.max(logits, axis=1, keepdims=True)
    cand = jnp.where(logits == bmax, col, jnp.int32(2**31 - 1))
    bidx = jnp.min(cand, axis=1, keepdims=True)

    m_old = m_ref[...]
    new_m = jnp.maximum(m_old, bmax)
    e = jnp.exp(logits - new_m)
    scale = jnp.exp(m_old - new_m)
    s_ref[...] = s_ref[...] * scale + jnp.sum(e, axis=1, keepdims=True)
    t_ref[...] = t_ref[...] * scale + jnp.sum(logits * e, axis=1, keepdims=True)
    m_ref[...] = new_m
    better = (bmax > m_old) | ((bmax == m_old) & (bidx < idx_ref[...]))
    idx_ref[...] = jnp.where(better, bidx, idx_ref[...])

    @pl.when(i == nb - 1)
    def _fin():
        m = m_ref[...]
        s = s_ref[...]
        lse = m + jnp.log(s)
        ns_ref[...] = idx_ref[...]
        lp_ref[...] = m - lse
        ent_ref[...] = lse - t_ref[...] / s


@jax.jit
def kernel(message, W, b):
    nb = pl.cdiv(VOCAB, BV)
    wt = W.T
    ns, lp, ent = pl.pallas_call(
        _body,
        grid=(nb,),
        in_specs=[
            pl.BlockSpec((B, D_MODEL), lambda i: (0, 0)),
            pl.BlockSpec((BV, D_MODEL), lambda i: (_blk(i), 0)),
            pl.BlockSpec((BV,), lambda i: (_blk(i),)),
        ],
        out_specs=[
            pl.BlockSpec((B, 1), lambda i: (0, 0)),
            pl.BlockSpec((B, 1), lambda i: (0, 0)),
            pl.BlockSpec((B, 1), lambda i: (0, 0)),
        ],
        out_shape=[
            jax.ShapeDtypeStruct((B, 1), jnp.int32),
            jax.ShapeDtypeStruct((B, 1), jnp.float32),
            jax.ShapeDtypeStruct((B, 1), jnp.float32),
        ],
        scratch_shapes=[
            pltpu.VMEM((B, 1), jnp.float32),
            pltpu.VMEM((B, 1), jnp.float32),
            pltpu.VMEM((B, 1), jnp.float32),
            pltpu.VMEM((B, 1), jnp.int32),
        ],
    )(message, wt, b)
    return ns[:, 0], lp[:, 0], ent[:, 0]
